# baseline (speedup 1.0000x reference)
import functools
import jax
import jax.numpy as jnp
from jax import lax
from jax.experimental import pallas as pl
from jax.experimental.pallas import tpu as pltpu
from jax.experimental.pallas import tpu_sc as plsc

E = 8
D = 1024
T = 2048
K = 256
NT = T // K
NU = NT + E - 1
NC = 2
NS = 16
NW = NC * NS
RPW = T // NW


def _routing_kernel(x_ref, wg_ref, perm_ref, gate_ref, counts_ref):
    x = x_ref[...]
    logits = jnp.dot(x, wg_ref[...].T, preferred_element_type=jnp.float32)
    m = jnp.max(logits, axis=1, keepdims=True)
    p = jnp.exp(logits - m)
    gates = p / jnp.sum(p, axis=1, keepdims=True)
    a_idx = jnp.argmax(gates, axis=1, keepdims=True).astype(jnp.int32)
    gmax = jnp.max(gates, axis=1, keepdims=True)
    gate_ref[...] = jnp.broadcast_to(gmax, (T, 128))

    iota_e = lax.broadcasted_iota(jnp.int32, (1, E), 1)
    oh = (a_idx == iota_e).astype(jnp.float32)

    r = lax.broadcasted_iota(jnp.int32, (K, K), 0)
    c = lax.broadcasted_iota(jnp.int32, (K, K), 1)
    tril = (c < r).astype(jnp.float32)
    blocks = []
    sums = []
    for b in range(NT):
        ob = oh[b * K:(b + 1) * K, :]
        blocks.append(jnp.dot(tril, ob, preferred_element_type=jnp.float32))
        sums.append(jnp.sum(ob, axis=0, keepdims=True))
    bsums = jnp.concatenate(sums, axis=0)
    r8 = lax.broadcasted_iota(jnp.int32, (NT, NT), 0)
    c8 = lax.broadcasted_iota(jnp.int32, (NT, NT), 1)
    tril8 = (c8 < r8).astype(jnp.float32)
    bpref = jnp.dot(tril8, bsums, preferred_element_type=jnp.float32)
    rank = jnp.concatenate(
        [blocks[b] + bpref[b:b + 1, :] for b in range(NT)], axis=0)

    counts = jnp.sum(bsums, axis=0, keepdims=True)
    zero1 = jnp.zeros((1, 1), jnp.float32)
    off = jnp.concatenate([zero1, counts[:, :E - 1]], axis=1)
    off = off + jnp.concatenate(
        [jnp.zeros((1, 1), jnp.float32), off[:, :E - 1]], axis=1)
    off = off + jnp.concatenate(
        [jnp.zeros((1, 2), jnp.float32), off[:, :E - 2]], axis=1)
    off = off + jnp.concatenate(
        [jnp.zeros((1, 4), jnp.float32), off[:, :E - 4]], axis=1)

    pos = jnp.sum((rank + off) * oh, axis=1, keepdims=True)
    perm_ref[...] = pos.astype(jnp.int32)
    counts_ref[...] = counts.astype(jnp.int32)


def _routing(x2, wg):
    return pl.pallas_call(
        _routing_kernel,
        in_specs=[
            pl.BlockSpec((T, D), lambda: (0, 0)),
            pl.BlockSpec((E, D), lambda: (0, 0)),
        ],
        out_specs=[
            pl.BlockSpec((T, 1), lambda: (0, 0)),
            pl.BlockSpec((T, 128), lambda: (0, 0)),
            pl.BlockSpec((1, E), lambda: (0, 0)),
        ],
        out_shape=[
            jax.ShapeDtypeStruct((T, 1), jnp.int32),
            jax.ShapeDtypeStruct((T, 128), jnp.float32),
            jax.ShapeDtypeStruct((1, E), jnp.int32),
        ],
    )(x2, wg)


_sc_mesh = plsc.VectorSubcoreMesh(core_axis_name="c", subcore_axis_name="s")


@functools.partial(
    pl.kernel, mesh=_sc_mesh,
    out_type=[
        jax.ShapeDtypeStruct((T, D), jnp.float32),
        jax.ShapeDtypeStruct((T, 128), jnp.float32),
    ],
    scratch_types=[
        pltpu.VMEM((RPW,), jnp.int32),
        pltpu.VMEM((RPW, D), jnp.float32),
        pltpu.VMEM((RPW, 128), jnp.float32),
        pltpu.SemaphoreType.DMA,
        pltpu.SemaphoreType.DMA,
    ],
)
def _dispatch(x_hbm, g_hbm, p_hbm, xs_hbm, gs_hbm, idx_v, rows_v, g_v,
              sem1, sem2):
    wid = lax.axis_index("s") * NC + lax.axis_index("c")
    base = wid * RPW
    pltpu.sync_copy(p_hbm.at[pl.ds(base, RPW)], idx_v)
    pltpu.sync_copy(x_hbm.at[pl.ds(base, RPW)], rows_v)
    pltpu.sync_copy(g_hbm.at[pl.ds(base, RPW)], g_v)
    cp1 = pltpu.async_copy(rows_v, xs_hbm.at[idx_v], sem1)
    cp2 = pltpu.async_copy(g_v, gs_hbm.at[idx_v], sem2)
    cp1.wait()
    cp2.wait()


def _gmm_kernel(tm, em, fm, om, xs_ref, gs_ref, We_ref, be_ref, out_ref):
    w = pl.program_id(0)
    e = em[w]
    t = tm[w]
    rows = t * K + lax.broadcasted_iota(jnp.int32, (K, 1), 0)
    mask = (rows >= om[e]) & (rows < om[e + 1])
    acc = jnp.dot(xs_ref[...], We_ref[0].T, preferred_element_type=jnp.float32)
    sub = (acc + be_ref[0, 0][None, :]) * gs_ref[:, 0:1]

    @pl.when(fm[w] == 1)
    def _():
        out_ref[...] = jnp.where(mask, sub, jnp.zeros_like(sub))

    @pl.when(fm[w] == 0)
    def _():
        out_ref[...] = jnp.where(mask, sub, out_ref[...])


def _gmm(xs, gs, We, be3, tm, em, fm, om):
    grid_spec = pltpu.PrefetchScalarGridSpec(
        num_scalar_prefetch=4,
        grid=(NU,),
        in_specs=[
            pl.BlockSpec((K, D), lambda w, tm, em, fm, om: (tm[w], 0)),
            pl.BlockSpec((K, 128), lambda w, tm, em, fm, om: (tm[w], 0)),
            pl.BlockSpec((1, D, D), lambda w, tm, em, fm, om: (em[w], 0, 0)),
            pl.BlockSpec((1, 1, D), lambda w, tm, em, fm, om: (em[w], 0, 0)),
        ],
        out_specs=pl.BlockSpec((K, D), lambda w, tm, em, fm, om: (tm[w], 0)),
    )
    return pl.pallas_call(
        _gmm_kernel,
        grid_spec=grid_spec,
        out_shape=jax.ShapeDtypeStruct((T, D), jnp.float32),
        compiler_params=pltpu.CompilerParams(
            dimension_semantics=("arbitrary",)),
    )(tm, em, fm, om, xs, gs, We, be3)


@functools.partial(
    pl.kernel, mesh=_sc_mesh,
    out_type=jax.ShapeDtypeStruct((T, D), jnp.float32),
    scratch_types=[
        pltpu.VMEM((RPW,), jnp.int32),
        pltpu.VMEM((RPW, D), jnp.float32),
        pltpu.SemaphoreType.DMA,
    ],
)
def _combine(ys_hbm, p_hbm, out_hbm, idx_v, rows_v, sem):
    wid = lax.axis_index("s") * NC + lax.axis_index("c")
    base = wid * RPW
    pltpu.sync_copy(p_hbm.at[pl.ds(base, RPW)], idx_v)
    pltpu.async_copy(ys_hbm.at[idx_v], rows_v, sem).wait()
    pltpu.sync_copy(rows_v, out_hbm.at[pl.ds(base, RPW)])


def kernel(x, wg, We, be):
    orig_shape = x.shape
    x2 = x.reshape(T, D)

    perm2, gate16, counts2 = _routing(x2, wg)
    perm = perm2.reshape(T)

    c = counts2.reshape(E)
    off = jnp.concatenate([jnp.zeros((1,), jnp.int32), jnp.cumsum(c)])
    t_lo = off[:E] // K
    t_hi = (jnp.maximum(off[1:], 1) - 1) // K
    num = jnp.where(c > 0, t_hi - t_lo + 1, 0)
    start = jnp.concatenate([jnp.zeros((1,), jnp.int32),
                             jnp.cumsum(num)]).astype(jnp.int32)
    total = start[E]
    w_idx = jnp.minimum(jnp.arange(NU, dtype=jnp.int32), total - 1)
    e_of_w = (jnp.searchsorted(start, w_idx, side="right") - 1).astype(jnp.int32)
    t_of_w = (t_lo[e_of_w] + (w_idx - start[e_of_w])).astype(jnp.int32)
    first_w = jnp.concatenate(
        [jnp.ones((1,), jnp.int32),
         (t_of_w[1:] != t_of_w[:-1]).astype(jnp.int32)])
    off = off.astype(jnp.int32)

    xs, gs = _dispatch(x2, gate16, perm)
    ys = _gmm(xs, gs, We, be.reshape(E, 1, D), t_of_w, e_of_w, first_w, off)
    out = _combine(ys, perm)
    return out.reshape(orig_shape)
